# Initial kernel scaffold; baseline (speedup 1.0000x reference)
import jax
import jax.numpy as jnp
from jax.experimental import pallas as pl


def kernel(X_2, indices):
    raise NotImplementedError("write your pallas kernel here")


# speedup vs baseline: 1.9524x; 1.9524x over previous
import functools

import jax
import jax.numpy as jnp
from jax import lax
from jax.experimental import pallas as pl
from jax.experimental.pallas import tpu as pltpu
from jax.experimental.pallas import tpu_sc as plsc

_D = 128
_CHUNK = 128
_NC = 2
_NS = 16
_NW = _NC * _NS


@functools.partial(jax.jit, static_argnames=("n_chunks",))
def _gather_rows(idx2d, table, n_chunks):
    mesh = plsc.VectorSubcoreMesh(
        core_axis_name="c", subcore_axis_name="s",
        num_cores=_NC, num_subcores=_NS,
    )

    base = n_chunks // _NW
    rem = n_chunks % _NW

    @functools.partial(
        pl.kernel,
        out_type=jax.ShapeDtypeStruct((n_chunks * _CHUNK, _D), jnp.float32),
        mesh=mesh,
        scratch_types=[
            pltpu.VMEM((_CHUNK,), jnp.int32),
            pltpu.VMEM((_CHUNK, _D), jnp.float32),
            pltpu.SemaphoreType.DMA,
        ],
    )
    def run(idx_hbm, table_hbm, out_hbm, idx_v, rows_v, sem):
        wid = lax.axis_index("c") * _NS + lax.axis_index("s")
        n_mine = base + jnp.where(wid < rem, 1, 0)

        def step(i, carry):
            chunk = wid + i * _NW
            pltpu.sync_copy(idx_hbm.at[chunk], idx_v)
            pltpu.async_copy(table_hbm.at[idx_v], rows_v, sem).wait()
            pltpu.sync_copy(rows_v, out_hbm.at[pl.ds(chunk * _CHUNK, _CHUNK)])
            return carry

        lax.fori_loop(0, n_mine, step, 0)

    return run(idx2d, table)


def kernel(X_2, indices):
    E = indices.shape[0]
    flat = indices.astype(jnp.int32).reshape(-1)
    n_chunks = flat.shape[0] // _CHUNK
    idx2d = flat.reshape(n_chunks, _CHUNK)
    rows = _gather_rows(idx2d, X_2, n_chunks)
    return rows.reshape(E, 2 * _D)


# speedup vs baseline: 2.4975x; 1.2792x over previous
import functools

import jax
import jax.numpy as jnp
from jax import lax
from jax.experimental import pallas as pl
from jax.experimental.pallas import tpu as pltpu
from jax.experimental.pallas import tpu_sc as plsc

_D = 128
_C = 100
_G = 4
_GC = _G * _C
_NC = 2
_NS = 16
_NW = _NC * _NS


@functools.partial(jax.jit, static_argnames=("n_chunks",))
def _gather_rows(idx2d, table, n_chunks):
    assert n_chunks % (_NW * 2 * _G) == 0
    w_chunks = n_chunks // _NW
    n_groups = w_chunks // _G
    t_iters = n_groups // 2
    w_rows = w_chunks * _C

    mesh = plsc.VectorSubcoreMesh(
        core_axis_name="c", subcore_axis_name="s",
        num_cores=_NC, num_subcores=_NS,
    )

    @functools.partial(
        pl.kernel,
        out_type=jax.ShapeDtypeStruct((n_chunks * _C, _D), jnp.float32),
        mesh=mesh,
        scratch_types=[
            pltpu.VMEM((w_chunks, _C), jnp.int32),
            pltpu.VMEM((2, _GC, _D), jnp.float32),
            pltpu.SemaphoreType.DMA,
            pltpu.SemaphoreType.DMA,
        ],
    )
    def run(idx_hbm, table_hbm, out_hbm, idx_v, rows_v, gsem0, gsem1):
        wid = lax.axis_index("c") * _NS + lax.axis_index("s")
        row_base = wid * w_rows
        pltpu.sync_copy(idx_hbm.at[pl.ds(wid * w_chunks, w_chunks)], idx_v)

        def start_group(g, p, sem):
            for b in range(_G):
                pltpu.async_copy(
                    table_hbm.at[idx_v.at[g * _G + b]],
                    rows_v.at[p, pl.ds(b * _C, _C)],
                    sem,
                )

        def wait_group(p, sem):
            pltpu.make_async_copy(
                out_hbm.at[pl.ds(0, _GC)], rows_v.at[p], sem
            ).wait()

        def write_group(g, p):
            pltpu.sync_copy(
                rows_v.at[p], out_hbm.at[pl.ds(row_base + g * _GC, _GC)]
            )

        start_group(0, 0, gsem0)

        def body(j, carry):
            g0 = 2 * j
            start_group(g0 + 1, 1, gsem1)
            wait_group(0, gsem0)
            write_group(g0, 0)

            @pl.when(j < t_iters - 1)
            def _():
                start_group(g0 + 2, 0, gsem0)

            wait_group(1, gsem1)
            write_group(g0 + 1, 1)
            return carry

        lax.fori_loop(0, t_iters, body, 0)

    return run(idx2d, table)


def kernel(X_2, indices):
    E = indices.shape[0]
    flat = indices.astype(jnp.int32).reshape(-1)
    n_chunks = flat.shape[0] // _C
    idx2d = flat.reshape(n_chunks, _C)
    rows = _gather_rows(idx2d, X_2, n_chunks)
    return rows.reshape(E, 2 * _D)


# speedup vs baseline: 7.0708x; 2.8311x over previous
import functools

import jax
import jax.numpy as jnp
from jax import lax
from jax.experimental import pallas as pl
from jax.experimental.pallas import tpu as pltpu
from jax.experimental.pallas import tpu_sc as plsc

_D = 128
_C = 40
_G = 5
_GC = _G * _C
_NC = 2
_NS = 16
_NW = _NC * _NS


@functools.partial(jax.jit, static_argnames=("n_edges",))
def _link_embed(src_idx, dst_idx, table, n_edges):
    assert n_edges % (_NW * 2 * _GC) == 0
    w_edges = n_edges // _NW
    n_groups = w_edges // _GC
    t_iters = n_groups // 2

    mesh = plsc.VectorSubcoreMesh(
        core_axis_name="c", subcore_axis_name="s",
        num_cores=_NC, num_subcores=_NS,
    )

    @functools.partial(
        pl.kernel,
        out_type=jax.ShapeDtypeStruct((n_edges, 2 * _D), jnp.float32),
        mesh=mesh,
        scratch_types=[
            pltpu.VMEM((w_edges,), jnp.int32),
            pltpu.VMEM((w_edges,), jnp.int32),
            pltpu.VMEM((2, _GC, _D), jnp.float32),
            pltpu.VMEM((2, _GC, _D), jnp.float32),
            pltpu.SemaphoreType.DMA,
            pltpu.SemaphoreType.DMA,
        ],
    )
    def run(src_hbm, dst_hbm, table_hbm, out_hbm,
            src_v, dst_v, srows_v, drows_v, gsem0, gsem1):
        wid = lax.axis_index("c") * _NS + lax.axis_index("s")
        edge_base = wid * w_edges
        pltpu.sync_copy(src_hbm.at[pl.ds(edge_base, w_edges)], src_v)
        pltpu.sync_copy(dst_hbm.at[pl.ds(edge_base, w_edges)], dst_v)

        def start_group(g, p, sem):
            for b in range(_G):
                off = g * _GC + b * _C
                pltpu.async_copy(
                    table_hbm.at[src_v.at[pl.ds(off, _C)]],
                    srows_v.at[p, pl.ds(b * _C, _C)],
                    sem,
                )
                pltpu.async_copy(
                    table_hbm.at[dst_v.at[pl.ds(off, _C)]],
                    drows_v.at[p, pl.ds(b * _C, _C)],
                    sem,
                )

        def wait_group(p, sem):
            pltpu.make_async_copy(
                table_hbm.at[pl.ds(0, _GC)], srows_v.at[p], sem
            ).wait()
            pltpu.make_async_copy(
                table_hbm.at[pl.ds(0, _GC)], drows_v.at[p], sem
            ).wait()

        def write_group(g, p):
            e0 = edge_base + g * _GC
            pltpu.sync_copy(
                srows_v.at[p], out_hbm.at[pl.ds(e0, _GC), pl.ds(0, _D)]
            )
            pltpu.sync_copy(
                drows_v.at[p], out_hbm.at[pl.ds(e0, _GC), pl.ds(_D, _D)]
            )

        start_group(0, 0, gsem0)

        def body(j, carry):
            g0 = 2 * j
            start_group(g0 + 1, 1, gsem1)
            wait_group(0, gsem0)
            write_group(g0, 0)

            @pl.when(j < t_iters - 1)
            def _():
                start_group(g0 + 2, 0, gsem0)

            wait_group(1, gsem1)
            write_group(g0 + 1, 1)
            return carry

        lax.fori_loop(0, t_iters, body, 0)

    return run(src_idx, dst_idx, table)


def kernel(X_2, indices):
    E = indices.shape[0]
    idx32 = indices.astype(jnp.int32)
    return _link_embed(idx32[:, 0], idx32[:, 1], X_2, E)


# speedup vs baseline: 10.8388x; 1.5329x over previous
import functools

import jax
import jax.numpy as jnp
from jax import lax
from jax.experimental import pallas as pl
from jax.experimental.pallas import tpu as pltpu
from jax.experimental.pallas import tpu_sc as plsc

_D = 128
_C = 40
_G = 1
_GC = _G * _C
_NC = 2
_NS = 16
_NW = _NC * _NS


@functools.partial(jax.jit, static_argnames=("n_edges",))
def _link_embed(src_idx, dst_idx, table, n_edges):
    assert n_edges % (_NW * 2 * _GC) == 0
    w_edges = n_edges // _NW
    n_groups = w_edges // _GC
    t_iters = n_groups // 2

    mesh = plsc.VectorSubcoreMesh(
        core_axis_name="c", subcore_axis_name="s",
        num_cores=_NC, num_subcores=_NS,
    )

    n_rows = table.shape[0]
    assert n_rows % (_NS * 8) == 0
    rows_per_tile = n_rows // _NS

    @functools.partial(
        pl.kernel,
        out_type=jax.ShapeDtypeStruct((n_edges, 2 * _D), jnp.float32),
        mesh=mesh,
        scratch_types=[
            pltpu.VMEM((w_edges,), jnp.int32),
            pltpu.VMEM((w_edges,), jnp.int32),
            pltpu.VMEM((2, _GC, _D), jnp.float32),
            pltpu.VMEM((2, _GC, _D), jnp.float32),
            pltpu.VMEM_SHARED((n_rows, _D), jnp.float32),
            pltpu.SemaphoreType.DMA,
            pltpu.SemaphoreType.DMA,
        ],
    )
    def run(src_hbm, dst_hbm, table_hbm, out_hbm,
            src_v, dst_v, srows_v, drows_v, table_sh, gsem0, gsem1):
        sid = lax.axis_index("s")
        wid = lax.axis_index("c") * _NS + sid
        edge_base = wid * w_edges
        r0 = sid * rows_per_tile
        pltpu.sync_copy(table_hbm.at[pl.ds(r0, rows_per_tile)],
                        table_sh.at[pl.ds(r0, rows_per_tile)])
        pltpu.sync_copy(src_hbm.at[pl.ds(edge_base, w_edges)], src_v)
        pltpu.sync_copy(dst_hbm.at[pl.ds(edge_base, w_edges)], dst_v)
        plsc.subcore_barrier()

        def start_group(g, p, sem):
            for b in range(_G):
                off = g * _GC + b * _C
                pltpu.async_copy(
                    table_sh.at[src_v.at[pl.ds(off, _C)]],
                    srows_v.at[p, pl.ds(b * _C, _C)],
                    sem,
                )
                pltpu.async_copy(
                    table_sh.at[dst_v.at[pl.ds(off, _C)]],
                    drows_v.at[p, pl.ds(b * _C, _C)],
                    sem,
                )

        def wait_group(p, sem):
            pltpu.make_async_copy(
                table_hbm.at[pl.ds(0, _GC)], srows_v.at[p], sem
            ).wait()
            pltpu.make_async_copy(
                table_hbm.at[pl.ds(0, _GC)], drows_v.at[p], sem
            ).wait()

        def write_group(g, p):
            e0 = edge_base + g * _GC
            pltpu.sync_copy(
                srows_v.at[p], out_hbm.at[pl.ds(e0, _GC), pl.ds(0, _D)]
            )
            pltpu.sync_copy(
                drows_v.at[p], out_hbm.at[pl.ds(e0, _GC), pl.ds(_D, _D)]
            )

        start_group(0, 0, gsem0)

        def body(j, carry):
            g0 = 2 * j
            start_group(g0 + 1, 1, gsem1)
            wait_group(0, gsem0)
            write_group(g0, 0)

            @pl.when(j < t_iters - 1)
            def _():
                start_group(g0 + 2, 0, gsem0)

            wait_group(1, gsem1)
            write_group(g0 + 1, 1)
            return carry

        lax.fori_loop(0, t_iters, body, 0)

    return run(src_idx, dst_idx, table)


def kernel(X_2, indices):
    E = indices.shape[0]
    idx32 = indices.astype(jnp.int32)
    pad = (-X_2.shape[0]) % (_NS * 8)
    table = jnp.pad(X_2, ((0, pad), (0, 0))) if pad else X_2
    return _link_embed(idx32[:, 0], idx32[:, 1], table, E)
